# baseline (speedup 1.0000x reference)
import functools

import jax
import jax.numpy as jnp
from jax import lax
from jax.experimental import pallas as pl
from jax.experimental.pallas import tpu as pltpu
from jax.experimental.pallas import tpu_sc as plsc

NN = 10000
NE = 320000
D = 128
DB = 16
NC = 2
NS = 16
NW = NC * NS
CH = 128
NCH = 80
EPW = CH * NCH
NE_P = NW * EPW
NN_P = 10240
RPW = NN_P // NS
GB = 2
NG = NCH // GB
CPW = EPW // CH
NCR = NE_P // CH


def _mesh():
    return plsc.VectorSubcoreMesh(core_axis_name="c", subcore_axis_name="s")


def _sc_gather(table, idx3):
    @functools.partial(
        pl.kernel,
        out_type=jax.ShapeDtypeStruct((NE_P, D), jnp.float32),
        mesh=_mesh(),
        scratch_types=[
            pltpu.VMEM((NCH, CH), jnp.int32),
            pltpu.VMEM((2, CH, D), jnp.float32),
            pltpu.VMEM_SHARED((NN_P, D), jnp.float32),
            pltpu.SemaphoreType.DMA((2,)),
        ],
    )
    def k(table_hbm, idx_hbm, out_hbm, idx_v, rows_v, tab_sh, gsem):
        c = lax.axis_index("c")
        s = lax.axis_index("s")
        w = s * NC + c
        pltpu.sync_copy(table_hbm.at[pl.ds(s * RPW, RPW)],
                        tab_sh.at[pl.ds(s * RPW, RPW)])
        pltpu.sync_copy(idx_hbm.at[w], idx_v)
        plsc.subcore_barrier()
        ebase = w * EPW

        def g_copy(j, b):
            return pltpu.make_async_copy(
                tab_sh.at[idx_v.at[j]], rows_v.at[b], gsem.at[b])

        g_copy(0, 0).start()
        g_copy(1, 1).start()

        def body(t, carry):
            j0 = 2 * t
            for b in range(2):
                j = j0 + b
                g_copy(j, b).wait()
                pltpu.sync_copy(rows_v.at[b],
                                out_hbm.at[pl.ds(ebase + j * CH, CH)])

                @pl.when(j + 2 < NCH)
                def _():
                    g_copy(j + 2, b).start()

            return carry

        lax.fori_loop(0, NCH // 2, body, 0)

    return k(table, idx3)


REV0 = 136
REV1 = 24


def _sc_scatter(vals3, dst3, zrows, rev2=None):
    with_rev = rev2 is not None
    out_types = [jax.ShapeDtypeStruct((NC, NN_P, D), jnp.float32)]
    if with_rev:
        out_types.append(jax.ShapeDtypeStruct((NE_P, D), jnp.float32))

    @functools.partial(
        pl.kernel,
        out_type=tuple(out_types),
        mesh=_mesh(),
        scratch_types=[
            pltpu.VMEM((NCH, CH), jnp.int32),
            pltpu.VMEM((2, CH, D), jnp.float32),
            pltpu.VMEM((2, CH), jnp.int32),
            pltpu.VMEM_SHARED((NN_P, D), jnp.float32),
            pltpu.SemaphoreType.DMA((2,)),
            pltpu.SemaphoreType.DMA((2,)),
            pltpu.SemaphoreType.DMA((2,)),
        ],
    )
    def k(vals_hbm, idx_hbm, z_hbm, *rest):
        if with_rev:
            (rev_hbm, out_hbm, rout_hbm, idx_v, rows_v, ridx_v, agg_sh,
             lsem, ssem, rsem) = rest
        else:
            (out_hbm, idx_v, rows_v, ridx_v, agg_sh,
             lsem, ssem, rsem) = rest
        c = lax.axis_index("c")
        s = lax.axis_index("s")
        w = s * NC + c
        pltpu.sync_copy(z_hbm.at[pl.ds(s * RPW, RPW)],
                        agg_sh.at[pl.ds(s * RPW, RPW)])
        pltpu.sync_copy(idx_hbm.at[w], idx_v)
        plsc.subcore_barrier()
        ebase = w * EPW

        def l_copy(j, b):
            return pltpu.make_async_copy(
                vals_hbm.at[pl.ds(ebase + j * CH, CH)], rows_v.at[b],
                lsem.at[b])

        def s_copy(j, b):
            return pltpu.make_async_copy(
                rows_v.at[b], agg_sh.at[idx_v.at[j]], ssem.at[b])

        l_copy(0, 0).start()
        l_copy(1, 1).start()

        def body(t, carry):
            j0 = 2 * t
            l_copy(j0, 0).wait()
            pltpu.async_copy(rows_v.at[0], agg_sh.at[idx_v.at[j0]],
                             ssem.at[0], add=True)
            l_copy(j0 + 1, 1).wait()
            pltpu.async_copy(rows_v.at[1], agg_sh.at[idx_v.at[j0 + 1]],
                             ssem.at[1], add=True)

            @pl.when(j0 + 2 < NCH)
            def _():
                s_copy(j0, 0).wait()
                l_copy(j0 + 2, 0).start()
                s_copy(j0 + 1, 1).wait()
                l_copy(j0 + 3, 1).start()

            return carry

        lax.fori_loop(0, NCH // 2, body, 0)
        s_copy(NCH - 2, 0).wait()
        s_copy(NCH - 1, 1).wait()
        plsc.subcore_barrier()
        pltpu.sync_copy(agg_sh.at[pl.ds(s * RPW, RPW)],
                        out_hbm.at[c, pl.ds(s * RPW, RPW)])

        if with_rev:
            start = jnp.where(c == 0, s * REV0, NS * REV0 + s * REV1)
            n_my = jnp.where(c == 0, REV0, REV1)

            def r_idx(i, b):
                return pltpu.make_async_copy(
                    rev_hbm.at[start + i], ridx_v.at[b], rsem.at[b])

            def r_gat(i, b):
                return pltpu.make_async_copy(
                    vals_hbm.at[ridx_v.at[b]], rows_v.at[b], lsem.at[b])

            r_idx(0, 0).start()
            r_idx(1, 1).start()

            def gbody(t, carry):
                i0 = 2 * t
                r_idx(i0, 0).wait()
                r_gat(i0, 0).start()
                r_idx(i0 + 1, 1).wait()
                r_gat(i0 + 1, 1).start()
                for b in range(2):
                    i = i0 + b
                    r_gat(i, b).wait()
                    pltpu.sync_copy(
                        rows_v.at[b],
                        rout_hbm.at[pl.ds((start + i) * CH, CH)])

                    @pl.when(i + 2 < n_my)
                    def _():
                        r_idx(i + 2, b).start()

                return carry

            lax.fori_loop(0, n_my // 2, gbody, 0)

    if with_rev:
        return k(vals3, dst3, zrows, rev2)
    return k(vals3, dst3, zrows)


def _tc_g(x_p, wix, bi2):
    BR = 512

    def body(x_ref, w_ref, b_ref, o_ref):
        o_ref[...] = jnp.dot(x_ref[...], w_ref[...],
                             preferred_element_type=jnp.float32) + b_ref[...]

    return pl.pallas_call(
        body,
        grid=(NN_P // BR,),
        in_specs=[
            pl.BlockSpec((BR, D), lambda i: (i, 0)),
            pl.BlockSpec((D, D), lambda i: (0, 0)),
            pl.BlockSpec((1, D), lambda i: (0, 0)),
        ],
        out_specs=pl.BlockSpec((BR, D), lambda i: (i, 0)),
        out_shape=jax.ShapeDtypeStruct((NN_P, D), jnp.float32),
    )(x_p, wix, bi2)


def _tc_h0p1(xs, ea, wie, wh):
    BR = 512

    def body(xs_ref, ea_ref, wie_ref, wh_ref, h0_ref, p_ref):
        h0 = xs_ref[...] + jnp.dot(ea_ref[...], wie_ref[...],
                                   preferred_element_type=jnp.float32)
        h0_ref[...] = h0.astype(jnp.bfloat16)
        p_ref[...] = jnp.dot(jnp.maximum(h0, 0.0), wh_ref[...],
                             preferred_element_type=jnp.float32)

    return pl.pallas_call(
        body,
        grid=(NE_P // BR,),
        in_specs=[
            pl.BlockSpec((BR, D), lambda i: (i, 0)),
            pl.BlockSpec((BR, DB), lambda i: (i, 0)),
            pl.BlockSpec((DB, D), lambda i: (0, 0)),
            pl.BlockSpec((D, D), lambda i: (0, 0)),
        ],
        out_specs=[pl.BlockSpec((BR, D), lambda i: (i, 0))] * 2,
        out_shape=[jax.ShapeDtypeStruct((NE_P, D), jnp.bfloat16),
                   jax.ShapeDtypeStruct((NE_P, D), jnp.float32)],
    )(xs, ea, wie, wh)


def _tc_addp(parts):
    BR = 512

    def body(p_ref, o_ref):
        o_ref[...] = p_ref[0] + p_ref[1]

    return pl.pallas_call(
        body,
        grid=(NN_P // BR,),
        in_specs=[pl.BlockSpec((NC, BR, D), lambda i: (0, i, 0))],
        out_specs=pl.BlockSpec((BR, D), lambda i: (i, 0)),
        out_shape=jax.ShapeDtypeStruct((NN_P, D), jnp.float32),
    )(parts)


def _tc_mm2(h0, s_g, r_g, bh2, wh):
    BR = 512

    def body(h0_ref, s_ref, r_ref, b_ref, wh_ref, p_ref):
        a = (h0_ref[...].astype(jnp.float32) + s_ref[...] - r_ref[...]
             + b_ref[...])
        p_ref[...] = jnp.dot(jnp.maximum(a, 0.0), wh_ref[...],
                             preferred_element_type=jnp.float32)

    return pl.pallas_call(
        body,
        grid=(NE_P // BR,),
        in_specs=[
            pl.BlockSpec((BR, D), lambda i: (i, 0)),
            pl.BlockSpec((BR, D), lambda i: (i, 0)),
            pl.BlockSpec((BR, D), lambda i: (i, 0)),
            pl.BlockSpec((1, D), lambda i: (0, 0)),
            pl.BlockSpec((D, D), lambda i: (0, 0)),
        ],
        out_specs=pl.BlockSpec((BR, D), lambda i: (i, 0)),
        out_shape=jax.ShapeDtypeStruct((NE_P, D), jnp.float32),
    )(h0, s_g, r_g, bh2, wh)


def _tc_h3(h0, s_g, r_g, bh2):
    BR = 512

    def body(h0_ref, s_ref, r_ref, b_ref, o_ref):
        a = (h0_ref[...].astype(jnp.float32) + s_ref[...] - r_ref[...]
             + b_ref[...])
        o_ref[...] = jnp.maximum(a, 0.0)

    return pl.pallas_call(
        body,
        grid=(NE_P // BR,),
        in_specs=[
            pl.BlockSpec((BR, D), lambda i: (i, 0)),
            pl.BlockSpec((BR, D), lambda i: (i, 0)),
            pl.BlockSpec((BR, D), lambda i: (i, 0)),
            pl.BlockSpec((1, D), lambda i: (0, 0)),
        ],
        out_specs=pl.BlockSpec((BR, D), lambda i: (i, 0)),
        out_shape=jax.ShapeDtypeStruct((NE_P, D), jnp.float32),
    )(h0, s_g, r_g, bh2)


def _tc_out(x, parts, wox, wom, bo2):
    BR = 400

    def body(x_ref, p_ref, wox_ref, wom_ref, b_ref, o_ref):
        aggf = p_ref[0] + p_ref[1]
        rs = jnp.sum(aggf, axis=1, keepdims=True)
        xb = x_ref[...]
        mfin = jnp.where(rs == 0.0, xb, aggf)
        acc = jnp.dot(xb, wox_ref[...], preferred_element_type=jnp.float32)
        acc += jnp.dot(mfin, wom_ref[...], preferred_element_type=jnp.float32)
        o_ref[...] = jnp.maximum(acc + b_ref[...], 0.0)

    return pl.pallas_call(
        body,
        grid=(NN // BR,),
        in_specs=[
            pl.BlockSpec((BR, D), lambda i: (i, 0)),
            pl.BlockSpec((NC, BR, D), lambda i: (0, i, 0)),
            pl.BlockSpec((D, D), lambda i: (0, 0)),
            pl.BlockSpec((D, D), lambda i: (0, 0)),
            pl.BlockSpec((1, D), lambda i: (0, 0)),
        ],
        out_specs=pl.BlockSpec((BR, D), lambda i: (i, 0)),
        out_shape=jax.ShapeDtypeStruct((NN, D), jnp.float32),
    )(x, parts, wox, wom, bo2)


def kernel(x, edge_index, rev_edge_index, edge_attr, Wi, bi, Wh, bh, Wo, bo):
    src = edge_index[0]
    dst = edge_index[1]
    pad = NE_P - NE
    src3 = jnp.pad(src, (0, pad)).reshape(NW, NCH, CH)
    rev2 = jnp.pad(rev_edge_index, (0, pad)).reshape(NCR, CH)
    dst3 = jnp.pad(dst, (0, pad), constant_values=NN_P - 1).reshape(NW, NCH, CH)
    ea_p = jnp.pad(edge_attr, ((0, pad), (0, 0)))
    zrows = jnp.zeros((NN_P, D), jnp.float32)
    wix, wie = Wi[:D], Wi[D:]
    wox, wom = Wo[:D], Wo[D:]
    bi2 = bi.reshape(1, D)
    bh2 = bh.reshape(1, D)
    bo2 = bo.reshape(1, D)

    x_p = jnp.pad(x, ((0, NN_P - NN), (0, 0)))
    g = _tc_g(x_p, wix, bi2)
    xs = _sc_gather(g, src3)
    h0, p = _tc_h0p1(xs, ea_p, wie, Wh)

    h3 = None
    for it in range(2):
        parts, r_g = _sc_scatter(p, dst3, zrows, rev2)
        agg = _tc_addp(parts)
        s_g = _sc_gather(agg, src3)
        if it == 0:
            p = _tc_mm2(h0, s_g, r_g, bh2, Wh)
        else:
            h3 = _tc_h3(h0, s_g, r_g, bh2)

    parts, = _sc_scatter(h3, dst3, zrows)
    return _tc_out(x, parts, wox, wom, bo2)


# speedup vs baseline: 1.0185x; 1.0185x over previous
import functools

import jax
import jax.numpy as jnp
from jax import lax
from jax.experimental import pallas as pl
from jax.experimental.pallas import tpu as pltpu
from jax.experimental.pallas import tpu_sc as plsc

NN = 10000
NE = 320000
D = 128
DB = 16
NC = 2
NS = 16
NW = NC * NS
CH = 128
NCH = 80
EPW = CH * NCH
NE_P = NW * EPW
NN_P = 10240
RPW = NN_P // NS
GB = 2
NG = NCH // GB
CPW = EPW // CH
NCR = NE_P // CH


def _mesh():
    return plsc.VectorSubcoreMesh(core_axis_name="c", subcore_axis_name="s")


def _sc_gather(table, idx3):
    @functools.partial(
        pl.kernel,
        out_type=jax.ShapeDtypeStruct((NE_P, D), jnp.float32),
        mesh=_mesh(),
        scratch_types=[
            pltpu.VMEM((NCH, CH), jnp.int32),
            pltpu.VMEM((2, CH, D), jnp.float32),
            pltpu.VMEM_SHARED((NN_P, D), jnp.float32),
            pltpu.SemaphoreType.DMA((2,)),
        ],
    )
    def k(table_hbm, idx_hbm, out_hbm, idx_v, rows_v, tab_sh, gsem):
        c = lax.axis_index("c")
        s = lax.axis_index("s")
        w = s * NC + c
        pltpu.sync_copy(table_hbm.at[pl.ds(s * RPW, RPW)],
                        tab_sh.at[pl.ds(s * RPW, RPW)])
        pltpu.sync_copy(idx_hbm.at[w], idx_v)
        plsc.subcore_barrier()
        ebase = w * EPW

        def g_copy(j, b):
            return pltpu.make_async_copy(
                tab_sh.at[idx_v.at[j]], rows_v.at[b], gsem.at[b])

        g_copy(0, 0).start()
        g_copy(1, 1).start()

        def body(t, carry):
            j0 = 2 * t
            for b in range(2):
                j = j0 + b
                g_copy(j, b).wait()
                pltpu.sync_copy(rows_v.at[b],
                                out_hbm.at[pl.ds(ebase + j * CH, CH)])

                @pl.when(j + 2 < NCH)
                def _():
                    g_copy(j + 2, b).start()

            return carry

        lax.fori_loop(0, NCH // 2, body, 0)

    return k(table, idx3)


REV0 = 120
REV1 = 40


def _sc_scatter(vals3, dst3, zrows, rev2=None):
    with_rev = rev2 is not None
    out_types = [jax.ShapeDtypeStruct((NC, NN_P, D), jnp.float32)]
    if with_rev:
        out_types.append(jax.ShapeDtypeStruct((NE_P, D), jnp.float32))

    @functools.partial(
        pl.kernel,
        out_type=tuple(out_types),
        mesh=_mesh(),
        scratch_types=[
            pltpu.VMEM((NCH, CH), jnp.int32),
            pltpu.VMEM((2, CH, D), jnp.float32),
            pltpu.VMEM((2, CH), jnp.int32),
            pltpu.VMEM_SHARED((NN_P, D), jnp.float32),
            pltpu.SemaphoreType.DMA((2,)),
            pltpu.SemaphoreType.DMA((2,)),
            pltpu.SemaphoreType.DMA((2,)),
        ],
    )
    def k(vals_hbm, idx_hbm, z_hbm, *rest):
        if with_rev:
            (rev_hbm, out_hbm, rout_hbm, idx_v, rows_v, ridx_v, agg_sh,
             lsem, ssem, rsem) = rest
        else:
            (out_hbm, idx_v, rows_v, ridx_v, agg_sh,
             lsem, ssem, rsem) = rest
        c = lax.axis_index("c")
        s = lax.axis_index("s")
        w = s * NC + c
        pltpu.sync_copy(z_hbm.at[pl.ds(s * RPW, RPW)],
                        agg_sh.at[pl.ds(s * RPW, RPW)])
        pltpu.sync_copy(idx_hbm.at[w], idx_v)
        plsc.subcore_barrier()
        ebase = w * EPW

        def l_copy(j, b):
            return pltpu.make_async_copy(
                vals_hbm.at[pl.ds(ebase + j * CH, CH)], rows_v.at[b],
                lsem.at[b])

        def s_copy(j, b):
            return pltpu.make_async_copy(
                rows_v.at[b], agg_sh.at[idx_v.at[j]], ssem.at[b])

        l_copy(0, 0).start()
        l_copy(1, 1).start()

        def body(t, carry):
            j0 = 2 * t
            l_copy(j0, 0).wait()
            pltpu.async_copy(rows_v.at[0], agg_sh.at[idx_v.at[j0]],
                             ssem.at[0], add=True)
            l_copy(j0 + 1, 1).wait()
            pltpu.async_copy(rows_v.at[1], agg_sh.at[idx_v.at[j0 + 1]],
                             ssem.at[1], add=True)

            @pl.when(j0 + 2 < NCH)
            def _():
                s_copy(j0, 0).wait()
                l_copy(j0 + 2, 0).start()
                s_copy(j0 + 1, 1).wait()
                l_copy(j0 + 3, 1).start()

            return carry

        lax.fori_loop(0, NCH // 2, body, 0)
        s_copy(NCH - 2, 0).wait()
        s_copy(NCH - 1, 1).wait()
        plsc.subcore_barrier()
        pltpu.sync_copy(agg_sh.at[pl.ds(s * RPW, RPW)],
                        out_hbm.at[c, pl.ds(s * RPW, RPW)])

        if with_rev:
            start = jnp.where(c == 0, s * REV0, NS * REV0 + s * REV1)
            n_my = jnp.where(c == 0, REV0, REV1)

            def r_idx(i, b):
                return pltpu.make_async_copy(
                    rev_hbm.at[start + i], ridx_v.at[b], rsem.at[b])

            def r_gat(i, b):
                return pltpu.make_async_copy(
                    vals_hbm.at[ridx_v.at[b]], rows_v.at[b], lsem.at[b])

            r_idx(0, 0).start()
            r_idx(1, 1).start()

            def gbody(t, carry):
                i0 = 2 * t
                r_idx(i0, 0).wait()
                r_gat(i0, 0).start()
                r_idx(i0 + 1, 1).wait()
                r_gat(i0 + 1, 1).start()
                for b in range(2):
                    i = i0 + b
                    r_gat(i, b).wait()
                    pltpu.sync_copy(
                        rows_v.at[b],
                        rout_hbm.at[pl.ds((start + i) * CH, CH)])

                    @pl.when(i + 2 < n_my)
                    def _():
                        r_idx(i + 2, b).start()

                return carry

            lax.fori_loop(0, n_my // 2, gbody, 0)

    if with_rev:
        return k(vals3, dst3, zrows, rev2)
    return k(vals3, dst3, zrows)


def _tc_g(x_p, wix, bi2):
    BR = 512

    def body(x_ref, w_ref, b_ref, o_ref):
        o_ref[...] = jnp.dot(x_ref[...], w_ref[...],
                             preferred_element_type=jnp.float32) + b_ref[...]

    return pl.pallas_call(
        body,
        grid=(NN_P // BR,),
        in_specs=[
            pl.BlockSpec((BR, D), lambda i: (i, 0)),
            pl.BlockSpec((D, D), lambda i: (0, 0)),
            pl.BlockSpec((1, D), lambda i: (0, 0)),
        ],
        out_specs=pl.BlockSpec((BR, D), lambda i: (i, 0)),
        out_shape=jax.ShapeDtypeStruct((NN_P, D), jnp.float32),
    )(x_p, wix, bi2)


def _tc_h0p1(xs, ea, wie, wh):
    BR = 512

    def body(xs_ref, ea_ref, wie_ref, wh_ref, h0_ref, p_ref):
        h0 = xs_ref[...] + jnp.dot(ea_ref[...], wie_ref[...],
                                   preferred_element_type=jnp.float32)
        h0_ref[...] = h0.astype(jnp.bfloat16)
        p_ref[...] = jnp.dot(jnp.maximum(h0, 0.0), wh_ref[...],
                             preferred_element_type=jnp.float32)

    return pl.pallas_call(
        body,
        grid=(NE_P // BR,),
        in_specs=[
            pl.BlockSpec((BR, D), lambda i: (i, 0)),
            pl.BlockSpec((BR, DB), lambda i: (i, 0)),
            pl.BlockSpec((DB, D), lambda i: (0, 0)),
            pl.BlockSpec((D, D), lambda i: (0, 0)),
        ],
        out_specs=[pl.BlockSpec((BR, D), lambda i: (i, 0))] * 2,
        out_shape=[jax.ShapeDtypeStruct((NE_P, D), jnp.bfloat16),
                   jax.ShapeDtypeStruct((NE_P, D), jnp.float32)],
    )(xs, ea, wie, wh)


def _tc_addp(parts):
    BR = 512

    def body(p_ref, o_ref):
        o_ref[...] = p_ref[0] + p_ref[1]

    return pl.pallas_call(
        body,
        grid=(NN_P // BR,),
        in_specs=[pl.BlockSpec((NC, BR, D), lambda i: (0, i, 0))],
        out_specs=pl.BlockSpec((BR, D), lambda i: (i, 0)),
        out_shape=jax.ShapeDtypeStruct((NN_P, D), jnp.float32),
    )(parts)


def _tc_mm2(h0, s_g, r_g, bh2, wh):
    BR = 512

    def body(h0_ref, s_ref, r_ref, b_ref, wh_ref, p_ref):
        a = (h0_ref[...].astype(jnp.float32) + s_ref[...] - r_ref[...]
             + b_ref[...])
        p_ref[...] = jnp.dot(jnp.maximum(a, 0.0), wh_ref[...],
                             preferred_element_type=jnp.float32)

    return pl.pallas_call(
        body,
        grid=(NE_P // BR,),
        in_specs=[
            pl.BlockSpec((BR, D), lambda i: (i, 0)),
            pl.BlockSpec((BR, D), lambda i: (i, 0)),
            pl.BlockSpec((BR, D), lambda i: (i, 0)),
            pl.BlockSpec((1, D), lambda i: (0, 0)),
            pl.BlockSpec((D, D), lambda i: (0, 0)),
        ],
        out_specs=pl.BlockSpec((BR, D), lambda i: (i, 0)),
        out_shape=jax.ShapeDtypeStruct((NE_P, D), jnp.float32),
    )(h0, s_g, r_g, bh2, wh)


def _tc_h3(h0, s_g, r_g, bh2):
    BR = 512

    def body(h0_ref, s_ref, r_ref, b_ref, o_ref):
        a = (h0_ref[...].astype(jnp.float32) + s_ref[...] - r_ref[...]
             + b_ref[...])
        o_ref[...] = jnp.maximum(a, 0.0)

    return pl.pallas_call(
        body,
        grid=(NE_P // BR,),
        in_specs=[
            pl.BlockSpec((BR, D), lambda i: (i, 0)),
            pl.BlockSpec((BR, D), lambda i: (i, 0)),
            pl.BlockSpec((BR, D), lambda i: (i, 0)),
            pl.BlockSpec((1, D), lambda i: (0, 0)),
        ],
        out_specs=pl.BlockSpec((BR, D), lambda i: (i, 0)),
        out_shape=jax.ShapeDtypeStruct((NE_P, D), jnp.float32),
    )(h0, s_g, r_g, bh2)


def _tc_out(x, parts, wox, wom, bo2):
    BR = 400

    def body(x_ref, p_ref, wox_ref, wom_ref, b_ref, o_ref):
        aggf = p_ref[0] + p_ref[1]
        rs = jnp.sum(aggf, axis=1, keepdims=True)
        xb = x_ref[...]
        mfin = jnp.where(rs == 0.0, xb, aggf)
        acc = jnp.dot(xb, wox_ref[...], preferred_element_type=jnp.float32)
        acc += jnp.dot(mfin, wom_ref[...], preferred_element_type=jnp.float32)
        o_ref[...] = jnp.maximum(acc + b_ref[...], 0.0)

    return pl.pallas_call(
        body,
        grid=(NN // BR,),
        in_specs=[
            pl.BlockSpec((BR, D), lambda i: (i, 0)),
            pl.BlockSpec((NC, BR, D), lambda i: (0, i, 0)),
            pl.BlockSpec((D, D), lambda i: (0, 0)),
            pl.BlockSpec((D, D), lambda i: (0, 0)),
            pl.BlockSpec((1, D), lambda i: (0, 0)),
        ],
        out_specs=pl.BlockSpec((BR, D), lambda i: (i, 0)),
        out_shape=jax.ShapeDtypeStruct((NN, D), jnp.float32),
    )(x, parts, wox, wom, bo2)


def kernel(x, edge_index, rev_edge_index, edge_attr, Wi, bi, Wh, bh, Wo, bo):
    src = edge_index[0]
    dst = edge_index[1]
    pad = NE_P - NE
    src3 = jnp.pad(src, (0, pad)).reshape(NW, NCH, CH)
    rev2 = jnp.pad(rev_edge_index, (0, pad)).reshape(NCR, CH)
    dst3 = jnp.pad(dst, (0, pad), constant_values=NN_P - 1).reshape(NW, NCH, CH)
    ea_p = jnp.pad(edge_attr, ((0, pad), (0, 0)))
    zrows = jnp.zeros((NN_P, D), jnp.float32)
    wix, wie = Wi[:D], Wi[D:]
    wox, wom = Wo[:D], Wo[D:]
    bi2 = bi.reshape(1, D)
    bh2 = bh.reshape(1, D)
    bo2 = bo.reshape(1, D)

    x_p = jnp.pad(x, ((0, NN_P - NN), (0, 0)))
    g = _tc_g(x_p, wix, bi2)
    xs = _sc_gather(g, src3)
    h0, p = _tc_h0p1(xs, ea_p, wie, Wh)

    h3 = None
    for it in range(2):
        parts, r_g = _sc_scatter(p, dst3, zrows, rev2)
        agg = _tc_addp(parts)
        s_g = _sc_gather(agg, src3)
        if it == 0:
            p = _tc_mm2(h0, s_g, r_g, bh2, Wh)
        else:
            h3 = _tc_h3(h0, s_g, r_g, bh2)

    parts, = _sc_scatter(h3, dst3, zrows)
    return _tc_out(x, parts, wox, wom, bo2)
